# baseline (speedup 1.0000x reference)
import functools

import jax
import jax.numpy as jnp
from jax import lax
from jax.experimental import pallas as pl
from jax.experimental.pallas import tpu as pltpu
from jax.experimental.pallas import tpu_sc as plsc

_CHUNK = 128


def _sc_gather_body(uidx_hbm, iidx_hbm, umf_hbm, imf_hbm, umlp_hbm, imlp_hbm,
                    umf_out, imf_out, umlp_out, imlp_out,
                    uidx_v, iidx_v, umf_v, imf_v, umlp_v, imlp_v, sem,
                    *, nc):
    wid = lax.axis_index("s") * nc + lax.axis_index("c")
    n_chunk = uidx_v.shape[0]
    bpw = n_chunk * _CHUNK
    base = wid * bpw
    pltpu.sync_copy(uidx_hbm.at[wid], uidx_v)
    pltpu.sync_copy(iidx_hbm.at[wid], iidx_v)
    copies = []
    for j in range(n_chunk):
        dst = pl.ds(j * _CHUNK, _CHUNK)
        copies.append(pltpu.async_copy(umf_hbm.at[uidx_v.at[j]], umf_v.at[dst], sem))
        copies.append(pltpu.async_copy(imf_hbm.at[iidx_v.at[j]], imf_v.at[dst], sem))
        copies.append(pltpu.async_copy(umlp_hbm.at[uidx_v.at[j]], umlp_v.at[dst], sem))
        copies.append(pltpu.async_copy(imlp_hbm.at[iidx_v.at[j]], imlp_v.at[dst], sem))
    for c in copies:
        c.wait()
    pltpu.sync_copy(umf_v, umf_out.at[pl.ds(base, bpw)])
    pltpu.sync_copy(imf_v, imf_out.at[pl.ds(base, bpw)])
    pltpu.sync_copy(umlp_v, umlp_out.at[pl.ds(base, bpw)])
    pltpu.sync_copy(imlp_v, imlp_out.at[pl.ds(base, bpw)])


def _sc_gather(user, item, u_mf_table, i_mf_table, u_mlp_table, i_mlp_table):
    info = plsc.get_sparse_core_info()
    nc, ns = info.num_cores, info.num_subcores
    nw = nc * ns
    b = user.shape[0]
    n_chunk = b // (nw * _CHUNK)
    bpw = n_chunk * _CHUNK
    dmf = u_mf_table.shape[1]
    dmlp = u_mlp_table.shape[1]

    u3 = user.astype(jnp.int32).reshape(nw, n_chunk, _CHUNK)
    i3 = item.astype(jnp.int32).reshape(nw, n_chunk, _CHUNK)

    f32 = jnp.float32
    run = pl.kernel(
        functools.partial(_sc_gather_body, nc=nc),
        mesh=plsc.VectorSubcoreMesh(core_axis_name="c", subcore_axis_name="s"),
        compiler_params=pltpu.CompilerParams(use_tc_tiling_on_sc=False),
        out_type=(
            jax.ShapeDtypeStruct((b, dmf), f32),
            jax.ShapeDtypeStruct((b, dmf), f32),
            jax.ShapeDtypeStruct((b, dmlp), f32),
            jax.ShapeDtypeStruct((b, dmlp), f32),
        ),
        scratch_types=[
            pltpu.VMEM((n_chunk, _CHUNK), jnp.int32),
            pltpu.VMEM((n_chunk, _CHUNK), jnp.int32),
            pltpu.VMEM((bpw, dmf), f32),
            pltpu.VMEM((bpw, dmf), f32),
            pltpu.VMEM((bpw, dmlp), f32),
            pltpu.VMEM((bpw, dmlp), f32),
            pltpu.SemaphoreType.DMA,
        ],
    )
    return run(u3, i3, u_mf_table, i_mf_table, u_mlp_table, i_mlp_table)


def _tc_dense_body(umf_ref, imf_ref, umlp_ref, imlp_ref, w1u_ref, w1i_ref,
                   b1_ref, w2_ref, b2_ref, wpmf_ref, wph_ref, bp_ref, out_ref):
    f32 = jnp.float32
    h1 = jnp.dot(umlp_ref[...], w1u_ref[...], preferred_element_type=f32)
    h1 = h1 + jnp.dot(imlp_ref[...], w1i_ref[...], preferred_element_type=f32)
    h1 = jnp.maximum(h1 + b1_ref[...], 0.0)
    h2 = jnp.dot(h1, w2_ref[...], preferred_element_type=f32) + b2_ref[...]
    h2 = jnp.maximum(h2, 0.0)
    mf = umf_ref[...] * imf_ref[...]
    z = (jnp.sum(mf * wpmf_ref[...], axis=1, keepdims=True)
         + jnp.sum(h2 * wph_ref[...], axis=1, keepdims=True)
         + bp_ref[0, 0])
    out_ref[...] = 1.0 / (1.0 + jnp.exp(-z))


def _tc_dense(umf, imf, umlp, imlp, W1, b1, W2, b2, Wp, bp):
    b = umf.shape[0]
    dmf = umf.shape[1]
    dmlp = umlp.shape[1]
    h1d = W1.shape[1]
    h2d = W2.shape[1]
    n_blocks = 8
    bb = b // n_blocks

    w1u = W1[:dmlp, :]
    w1i = W1[dmlp:, :]
    b1r = b1.reshape(1, h1d)
    b2r = b2.reshape(1, h2d)
    wpmf = Wp[:dmf, 0].reshape(1, dmf)
    wph = Wp[dmf:, 0].reshape(1, h2d)
    bpr = bp.reshape(1, 1)

    row = lambda i: (i, 0)
    fix = lambda i: (0, 0)
    return pl.pallas_call(
        _tc_dense_body,
        grid=(n_blocks,),
        in_specs=[
            pl.BlockSpec((bb, dmf), row),
            pl.BlockSpec((bb, dmf), row),
            pl.BlockSpec((bb, dmlp), row),
            pl.BlockSpec((bb, dmlp), row),
            pl.BlockSpec((dmlp, h1d), fix),
            pl.BlockSpec((dmlp, h1d), fix),
            pl.BlockSpec((1, h1d), fix),
            pl.BlockSpec((h1d, h2d), fix),
            pl.BlockSpec((1, h2d), fix),
            pl.BlockSpec((1, dmf), fix),
            pl.BlockSpec((1, h2d), fix),
            pl.BlockSpec((1, 1), fix),
        ],
        out_specs=pl.BlockSpec((bb, 1), row),
        out_shape=jax.ShapeDtypeStruct((b, 1), jnp.float32),
    )(umf, imf, umlp, imlp, w1u, w1i, b1r, W2, b2r, wpmf, wph, bpr)


def kernel(user, item, u_mf_table, i_mf_table, u_mlp_table, i_mlp_table,
           W1, b1, W2, b2, Wp, bp):
    umf, imf, umlp, imlp = _sc_gather(
        user, item, u_mf_table, i_mf_table, u_mlp_table, i_mlp_table)
    return _tc_dense(umf, imf, umlp, imlp, W1, b1, W2, b2, Wp, bp)
